# baseline (speedup 1.0000x reference)
import jax
import jax.numpy as jnp
from jax import lax
from jax.experimental import pallas as pl
from jax.experimental.pallas import tpu as pltpu
from jax.experimental.pallas import tpu_sc as plsc

Q = 2048
K = 2048
D = 64
V = 513
MAXP = (V - 1) // 2
PLEN = Q + K - 1
LB = Q - 1 - MAXP

NC, NS = 2, 16
NW = NC * NS
GROUP = 4

SPLIT = 2048
SC_ROWS = Q - SPLIT
ROWS_PER = max(SC_ROWS // NW, 1)
RPC = 64


def _sc_body(emb_hbm, out_hbm, p_hbm, p_sh, sem):
    c = lax.axis_index("c")
    s = lax.axis_index("s")
    wid = s * NC + c

    @pl.when(s == 0)
    def _():
        pltpu.sync_copy(emb_hbm, p_sh.at[pl.ds(LB, V)])
        done = 1
        while done < LB + 1:
            m = min(done, LB + 1 - done)
            pltpu.sync_copy(p_sh.at[pl.ds(LB - m + 1, m)],
                            p_sh.at[pl.ds(LB - done - m + 1, m)])
            done += m
        R = LB + V - 1
        done = 1
        while done < LB + 1:
            m = min(done, LB + 1 - done)
            pltpu.sync_copy(p_sh.at[pl.ds(R, m)],
                            p_sh.at[pl.ds(R + done, m)])
            done += m

    @pl.when((s == 0) & (c == 0))
    def _():
        pltpu.sync_copy(p_sh, p_hbm)

    plsc.subcore_barrier()

    if SC_ROWS > 0:
        base_i = SPLIT + wid * ROWS_PER
        for g in range(ROWS_PER // GROUP if ROWS_PER >= GROUP else 0):
            cps = []
            for b in range(GROUP):
                i = base_i + g * GROUP + b
                cps.append(pltpu.async_copy(
                    p_sh.at[pl.ds((Q - 1) - i, K)], out_hbm.at[i], sem))
            for cp in cps:
                cp.wait()
        for r in range(ROWS_PER - (ROWS_PER // GROUP) * GROUP):
            i = base_i + (ROWS_PER // GROUP) * GROUP + r
            pltpu.async_copy(
                p_sh.at[pl.ds((Q - 1) - i, K)], out_hbm.at[i], sem).wait()


def _tc_body(p0_ref, p1_ref, prev_ref, out_ref, sem):
    del prev_ref
    base = pl.program_id(0) * RPC
    cps = []
    for b in range(RPC):
        i = base + b
        if b % 2 == 1:
            src = p0_ref.at[pl.ds(((Q - 1) - i) // 2, K * D // 128), :]
        else:
            src = p1_ref.at[pl.ds(((Q - 2) - i) // 2, K * D // 128), :]
        cps.append(pltpu.async_copy(src, out_ref.at[i], sem))
    for cp in cps:
        cp.wait()


def kernel(q, v, embeddings):
    del q, v
    mesh = plsc.VectorSubcoreMesh(core_axis_name="c", subcore_axis_name="s")
    out_sc, p_hbm = pl.kernel(
        _sc_body,
        mesh=mesh,
        out_type=[
            jax.ShapeDtypeStruct((Q, K, D), jnp.float32),
            jax.ShapeDtypeStruct((PLEN, D), jnp.float32),
        ],
        scratch_types=[
            pltpu.VMEM_SHARED((PLEN, D), jnp.float32),
            pltpu.SemaphoreType.DMA,
        ],
    )(embeddings)

    if SPLIT == 0:
        return out_sc

    p_flat = p_hbm.reshape(PLEN * D)
    p2 = p_flat[: (PLEN - 1) * D].reshape((PLEN - 1) // 2, 128)
    p2s = p_flat[D:].reshape((PLEN - 1) // 2, 128)
    out3 = out_sc.reshape(Q, K * D // 128, 128)
    res = pl.pallas_call(
        _tc_body,
        grid=(SPLIT // RPC,),
        in_specs=[
            pl.BlockSpec(((PLEN - 1) // 2, 128), lambda g: (0, 0)),
            pl.BlockSpec(((PLEN - 1) // 2, 128), lambda g: (0, 0)),
            pl.BlockSpec(memory_space=pltpu.MemorySpace.HBM),
        ],
        out_specs=pl.BlockSpec(memory_space=pltpu.MemorySpace.HBM),
        out_shape=jax.ShapeDtypeStruct((Q, K * D // 128, 128), jnp.float32),
        scratch_shapes=[pltpu.SemaphoreType.DMA],
        input_output_aliases={2: 0},
    )(p2, p2s, out3)
    return res.reshape(Q, K, D)


# speedup vs baseline: 2.0904x; 2.0904x over previous
import jax
import jax.numpy as jnp
from jax import lax
from jax.experimental import pallas as pl
from jax.experimental.pallas import tpu as pltpu
from jax.experimental.pallas import tpu_sc as plsc

Q = 2048
K = 2048
D = 64
V = 513
MAXP = (V - 1) // 2
PLEN = Q + K - 1
LB = Q - 1 - MAXP

NC, NS = 2, 16
NW = NC * NS
GROUP = 4

SPLIT = 1024
SC_ROWS = Q - SPLIT
ROWS_PER = SC_ROWS // NW if SC_ROWS else 0
RPC = 64


def _sc_body(emb_hbm, out_hbm, p_sh, sem):
    c = lax.axis_index("c")
    s = lax.axis_index("s")
    wid = s * NC + c

    @pl.when(s == 0)
    def _():
        pltpu.sync_copy(emb_hbm, p_sh.at[pl.ds(LB, V)])
        done = 1
        while done < LB + 1:
            m = min(done, LB + 1 - done)
            pltpu.sync_copy(p_sh.at[pl.ds(LB - m + 1, m)],
                            p_sh.at[pl.ds(LB - done - m + 1, m)])
            done += m
        R = LB + V - 1
        done = 1
        while done < LB + 1:
            m = min(done, LB + 1 - done)
            pltpu.sync_copy(p_sh.at[pl.ds(R, m)],
                            p_sh.at[pl.ds(R + done, m)])
            done += m

    plsc.subcore_barrier()

    for g in range(ROWS_PER // GROUP):
        cps = []
        for b in range(GROUP):
            r = wid * ROWS_PER + g * GROUP + b
            cps.append(pltpu.async_copy(
                p_sh.at[pl.ds((Q - 1) - SPLIT - r, K)], out_hbm.at[r], sem))
        for cp in cps:
            cp.wait()


def _tc_body(emb_ref, out_ref, p_vmem, sem):
    @pl.when(pl.program_id(0) == 0)
    def _():
        p_vmem[pl.ds(0, LB), :] = jnp.broadcast_to(emb_ref[0:1, :], (LB, D))
        p_vmem[pl.ds(LB, V), :] = emb_ref[...]
        p_vmem[pl.ds(LB + V, PLEN - LB - V), :] = jnp.broadcast_to(
            emb_ref[V - 1:V, :], (PLEN - LB - V, D))

    base = pl.program_id(0) * RPC
    cps = []
    for b in range(RPC):
        i = base + b
        cps.append(pltpu.async_copy(
            p_vmem.at[pl.ds((Q - 1) - i, K), :], out_ref.at[i], sem))
    for cp in cps:
        cp.wait()


def kernel(q, v, embeddings):
    del q, v
    mesh = plsc.VectorSubcoreMesh(core_axis_name="c", subcore_axis_name="s")
    out_sc = pl.kernel(
        _sc_body,
        mesh=mesh,
        out_type=jax.ShapeDtypeStruct((SC_ROWS, K, D), jnp.float32),
        scratch_types=[
            pltpu.VMEM_SHARED((PLEN, D), jnp.float32),
            pltpu.SemaphoreType.DMA,
        ],
    )(embeddings)

    out_tc = pl.pallas_call(
        _tc_body,
        grid=(SPLIT // RPC,),
        in_specs=[pl.BlockSpec((V, D), lambda g: (0, 0))],
        out_specs=pl.BlockSpec(memory_space=pltpu.MemorySpace.HBM),
        out_shape=jax.ShapeDtypeStruct((Q, K, D), jnp.float32),
        scratch_shapes=[
            pltpu.VMEM((PLEN, D), jnp.float32),
            pltpu.SemaphoreType.DMA,
        ],
    )(embeddings)

    return lax.dynamic_update_slice(out_tc, out_sc, (SPLIT, 0, 0))
